# baseline (speedup 1.0000x reference)
import functools

import jax
import jax.numpy as jnp
from jax import lax
from jax.experimental import pallas as pl
from jax.experimental.pallas import tpu as pltpu
from jax.experimental.pallas import tpu_sc as plsc

N_NODES = 10000
N_EDGES = 320000
D_FEAT = 128
NUM_GRAPHS = 64

NS = 16
SL = 656
N_SH = NS * SL
EC = 160
E_TILE = EC * 128
E_REAL = N_EDGES // NS
BC = 6
DUMMY_N = 10240
DUMMY_G = NUM_GRAPHS
G_SH = 128
MAGIC = 0x5F3759DF


def _matvec_body(x_ref, w_ref, o_ref):
    o_ref[...] = jnp.dot(x_ref[...], w_ref[...],
                         preferred_element_type=jnp.float32)


def _edge_round(u_sh, acc_sh, src_v, dst_v, ulocal, accp, iota_v):
    pltpu.sync_copy(u_sh, ulocal)

    def _zero(j, carry):
        accp[pl.ds(j * 16, 16)] = jnp.zeros((16,), jnp.float32)
        return carry

    lax.fori_loop(0, N_SH // 16, _zero, 0)

    def _gs(j, carry):
        for g in range(4):
            o = j * 64 + g * 16
            v = plsc.load_gather(ulocal, [src_v[pl.ds(o, 16)]])
            plsc.addupdate_scatter(accp, [dst_v[pl.ds(o, 16)]], v)
        return carry

    lax.fori_loop(0, E_TILE // 64, _gs, 0)
    pltpu.sync_copy(accp, acc_sh.at[iota_v], add=True)


def _sc_body(srcf_hbm, dstf_hbm, batch_hbm, y_hbm, pooled_hbm, cnt_hbm,
             src_v, dst_v, vals_v, batch_v, pvals_v, ulocal, accp, iota_v,
             ybuf, dinvbuf, abuf, ubuf,
             zbuf, sbuf, cbuf, obuf, ocnt,
             deg_sh, u_sh, acc1_sh, acc2_sh, sums_sh, counts_sh):
    s = lax.axis_index("s")
    c = lax.axis_index("c")
    base = s * SL

    pltpu.sync_copy(srcf_hbm.at[pl.ds(s * E_REAL, E_REAL)],
                    src_v.at[pl.ds(0, E_REAL)])
    pltpu.sync_copy(dstf_hbm.at[pl.ds(s * E_REAL, E_REAL)],
                    dst_v.at[pl.ds(0, E_REAL)])
    for i in range((E_TILE - E_REAL) // 16):
        src_v[pl.ds(E_REAL + i * 16, 16)] = jnp.full((16,), DUMMY_N,
                                                     jnp.int32)
        dst_v[pl.ds(E_REAL + i * 16, 16)] = jnp.full((16,), DUMMY_N,
                                                     jnp.int32)
    pltpu.sync_copy(batch_hbm.at[s], batch_v)
    pltpu.sync_copy(y_hbm.at[pl.ds(base, SL)], ybuf)

    for i in range(128 // 16):
        zbuf[pl.ds(i * 16, 16)] = jnp.zeros((16,), jnp.float32)
    for i in range(SL // 16):
        abuf[pl.ds(i * 16, 16)] = jnp.full((16,), 1.0, jnp.float32)
    pltpu.sync_copy(abuf, deg_sh.at[pl.ds(base, SL)])

    @pl.when(s == 0)
    def _init_graph_accs():
        pltpu.sync_copy(zbuf, sums_sh)
        pltpu.sync_copy(zbuf, counts_sh)

    plsc.subcore_barrier()

    def _fill_iota(j, carry):
        iota_v[pl.ds(j * 16, 16)] = (
            lax.iota(jnp.int32, 16) + jnp.int32(16) * j)
        accp[pl.ds(j * 16, 16)] = jnp.zeros((16,), jnp.float32)
        return carry

    lax.fori_loop(0, N_SH // 16, _fill_iota, 0)
    ones16 = jnp.full((16,), 1.0, jnp.float32)

    def _hist(j, carry):
        for g in range(4):
            idx = dst_v[pl.ds(j * 64 + g * 16, 16)]
            plsc.addupdate_scatter(accp, [idx], ones16)
        return carry

    lax.fori_loop(0, E_TILE // 64, _hist, 0)
    pltpu.sync_copy(accp, deg_sh.at[iota_v], add=True)

    def _fill_ones(j, carry):
        for g in range(8):
            vals_v[pl.ds(j * 128 + g * 16, 16)] = jnp.full((16,), 1.0,
                                                           jnp.float32)
        return carry

    lax.fori_loop(0, BC, _fill_ones, 0)
    pltpu.sync_copy(vals_v.at[pl.ds(0, BC * 128)],
                    counts_sh.at[batch_v], add=True)
    plsc.subcore_barrier()

    pltpu.sync_copy(deg_sh.at[pl.ds(base, SL)], abuf)
    for i in range(SL // 16):
        d = abuf[pl.ds(i * 16, 16)]
        ii = plsc.bitcast(d, jnp.int32)
        ii = jnp.int32(MAGIC) - lax.shift_right_logical(ii, 1)
        r = plsc.bitcast(ii, jnp.float32)
        for _ in range(3):
            r = r * (1.5 - 0.5 * d * r * r)
        dinvbuf[pl.ds(i * 16, 16)] = r
        ubuf[pl.ds(i * 16, 16)] = r * ybuf[pl.ds(i * 16, 16)]
    pltpu.sync_copy(ubuf, u_sh.at[pl.ds(base, SL)])
    pltpu.sync_copy(ubuf, acc1_sh.at[pl.ds(base, SL)])
    plsc.subcore_barrier()

    _edge_round(u_sh, acc1_sh, src_v, dst_v, ulocal, accp, iota_v)
    plsc.subcore_barrier()

    pltpu.sync_copy(acc1_sh.at[pl.ds(base, SL)], abuf)
    for i in range(SL // 16):
        r = dinvbuf[pl.ds(i * 16, 16)]
        ubuf[pl.ds(i * 16, 16)] = r * r * abuf[pl.ds(i * 16, 16)]
    pltpu.sync_copy(ubuf, u_sh.at[pl.ds(base, SL)])
    pltpu.sync_copy(ubuf, acc2_sh.at[pl.ds(base, SL)])
    plsc.subcore_barrier()

    _edge_round(u_sh, acc2_sh, src_v, dst_v, ulocal, accp, iota_v)
    plsc.subcore_barrier()

    pltpu.sync_copy(acc2_sh.at[pl.ds(base, SL)], abuf)
    for i in range(SL // 16):
        pvals_v[pl.ds(i * 16, 16)] = (
            dinvbuf[pl.ds(i * 16, 16)] * abuf[pl.ds(i * 16, 16)])

    pltpu.sync_copy(pvals_v, sums_sh.at[batch_v], add=True)
    plsc.subcore_barrier()

    @pl.when((s == 0) & (c == 0))
    def _finalize():
        pltpu.sync_copy(sums_sh, sbuf)
        pltpu.sync_copy(counts_sh, cbuf)
        for i in range(NUM_GRAPHS // 16):
            sv = sbuf[pl.ds(i * 16, 16)]
            cv = cbuf[pl.ds(i * 16, 16)]
            obuf[pl.ds(i * 16, 16)] = sv / jnp.maximum(cv, 1.0)
            ocnt[pl.ds(i * 16, 16)] = cv
        pltpu.sync_copy(obuf, pooled_hbm)
        pltpu.sync_copy(ocnt, cnt_hbm)


_sc_call = functools.partial(
    pl.kernel,
    out_type=(jax.ShapeDtypeStruct((NUM_GRAPHS,), jnp.float32),
              jax.ShapeDtypeStruct((NUM_GRAPHS,), jnp.float32)),
    mesh=plsc.VectorSubcoreMesh(core_axis_name="c", subcore_axis_name="s",
                                num_cores=1),
    compiler_params=pltpu.CompilerParams(needs_layout_passes=False),
    scratch_types=[
        pltpu.VMEM((E_TILE,), jnp.int32),
        pltpu.VMEM((E_TILE,), jnp.int32),
        pltpu.VMEM((BC * 128,), jnp.float32),
        pltpu.VMEM((BC * 128,), jnp.int32),
        pltpu.VMEM((BC * 128,), jnp.float32),
        pltpu.VMEM((N_SH,), jnp.float32),
        pltpu.VMEM((N_SH,), jnp.float32),
        pltpu.VMEM((N_SH,), jnp.int32),
        pltpu.VMEM((SL,), jnp.float32),
        pltpu.VMEM((SL,), jnp.float32),
        pltpu.VMEM((SL,), jnp.float32),
        pltpu.VMEM((SL,), jnp.float32),
        pltpu.VMEM((128,), jnp.float32),
        pltpu.VMEM((G_SH,), jnp.float32),
        pltpu.VMEM((G_SH,), jnp.float32),
        pltpu.VMEM((NUM_GRAPHS,), jnp.float32),
        pltpu.VMEM((NUM_GRAPHS,), jnp.float32),
        pltpu.VMEM_SHARED((N_SH,), jnp.float32),
        pltpu.VMEM_SHARED((N_SH,), jnp.float32),
        pltpu.VMEM_SHARED((N_SH,), jnp.float32),
        pltpu.VMEM_SHARED((N_SH,), jnp.float32),
        pltpu.VMEM_SHARED((G_SH,), jnp.float32),
        pltpu.VMEM_SHARED((G_SH,), jnp.float32),
    ],
)(_sc_body)


def kernel(x, edge_index, batch, W, b):
    y2d = pl.pallas_call(
        _matvec_body,
        out_shape=jax.ShapeDtypeStruct((N_NODES, 1), jnp.float32),
    )(x, W)
    y_pad = jnp.pad(y2d[:, 0], (0, N_SH - N_NODES))
    batch_p = jnp.pad(batch, (0, N_SH - N_NODES),
                      constant_values=DUMMY_G).reshape(NS, SL)
    batch_p = jnp.pad(batch_p, ((0, 0), (0, BC * 128 - SL)),
                      constant_values=DUMMY_G).reshape(NS, BC * 128)
    pooled, counts = _sc_call(edge_index[0], edge_index[1], batch_p, y_pad)
    out = pooled + b * (counts > 0.0)
    return out.reshape(NUM_GRAPHS, 1)


# speedup vs baseline: 1.0646x; 1.0646x over previous
import functools

import jax
import jax.numpy as jnp
from jax import lax
from jax.experimental import pallas as pl
from jax.experimental.pallas import tpu as pltpu
from jax.experimental.pallas import tpu_sc as plsc

N_NODES = 10000
N_EDGES = 320000
D_FEAT = 128
NUM_GRAPHS = 64

NS = 16
SL = 656
N_SH = NS * SL
EC = 160
E_TILE = EC * 128
E_REAL = N_EDGES // NS
BC = 6
DUMMY_N = 10240
DUMMY_G = NUM_GRAPHS
G_SH = 128
MAGIC = 0x5F3759DF


def _matvec_body(x_ref, w_ref, o_ref):
    o_ref[...] = jnp.dot(x_ref[...], w_ref[...],
                         preferred_element_type=jnp.float32)


def _edge_round(u_sh, acc_sh, src_v, dst_v, ulocal, accp, iota_v):
    pltpu.sync_copy(u_sh, ulocal)

    def _zero(j, carry):
        for g in range(4):
            accp[pl.ds(j * 64 + g * 16, 16)] = jnp.zeros((16,), jnp.float32)
        return carry

    lax.fori_loop(0, N_SH // 64, _zero, 0)

    def _gs(j, carry):
        for g in range(4):
            o = j * 64 + g * 16
            v = plsc.load_gather(ulocal, [src_v[pl.ds(o, 16)]])
            plsc.addupdate_scatter(accp, [dst_v[pl.ds(o, 16)]], v)
        return carry

    lax.fori_loop(0, E_TILE // 64, _gs, 0)
    pltpu.sync_copy(accp, acc_sh.at[iota_v], add=True)


def _sc_body(srcf_hbm, dstf_hbm, batch_hbm, y_hbm, pooled_hbm, cnt_hbm,
             src_v, dst_v, vals_v, batch_v, pvals_v, ulocal, accp, iota_v,
             ybuf, dinvbuf, abuf, ubuf,
             zbuf, sbuf, cbuf, obuf, ocnt,
             deg_sh, u_sh, acc1_sh, acc2_sh, sums_sh, counts_sh, sem):
    s = lax.axis_index("s")
    c = lax.axis_index("c")
    base = s * SL

    d_src = pltpu.async_copy(srcf_hbm.at[pl.ds(s * E_REAL, E_REAL)],
                             src_v.at[pl.ds(0, E_REAL)], sem)
    d_dst = pltpu.async_copy(dstf_hbm.at[pl.ds(s * E_REAL, E_REAL)],
                             dst_v.at[pl.ds(0, E_REAL)], sem)
    d_bat = pltpu.async_copy(batch_hbm.at[s], batch_v, sem)
    d_y = pltpu.async_copy(y_hbm.at[pl.ds(base, SL)], ybuf, sem)

    for i in range(128 // 16):
        zbuf[pl.ds(i * 16, 16)] = jnp.zeros((16,), jnp.float32)
    for i in range(SL // 16):
        abuf[pl.ds(i * 16, 16)] = jnp.full((16,), 1.0, jnp.float32)
    pltpu.sync_copy(abuf, deg_sh.at[pl.ds(base, SL)])

    d_src.wait()
    d_dst.wait()
    d_bat.wait()
    d_y.wait()
    for i in range((E_TILE - E_REAL) // 16):
        src_v[pl.ds(E_REAL + i * 16, 16)] = jnp.full((16,), DUMMY_N,
                                                     jnp.int32)
        dst_v[pl.ds(E_REAL + i * 16, 16)] = jnp.full((16,), DUMMY_N,
                                                     jnp.int32)

    @pl.when(s == 0)
    def _init_graph_accs():
        pltpu.sync_copy(zbuf, sums_sh)
        pltpu.sync_copy(zbuf, counts_sh)

    plsc.subcore_barrier()

    def _fill_iota(j, carry):
        iota_v[pl.ds(j * 16, 16)] = (
            lax.iota(jnp.int32, 16) + jnp.int32(16) * j)
        accp[pl.ds(j * 16, 16)] = jnp.zeros((16,), jnp.float32)
        return carry

    lax.fori_loop(0, N_SH // 16, _fill_iota, 0)
    ones16 = jnp.full((16,), 1.0, jnp.float32)

    def _hist(j, carry):
        for g in range(4):
            idx = dst_v[pl.ds(j * 64 + g * 16, 16)]
            plsc.addupdate_scatter(accp, [idx], ones16)
        return carry

    lax.fori_loop(0, E_TILE // 64, _hist, 0)
    pltpu.sync_copy(accp, deg_sh.at[iota_v], add=True)

    def _fill_ones(j, carry):
        for g in range(8):
            vals_v[pl.ds(j * 128 + g * 16, 16)] = jnp.full((16,), 1.0,
                                                           jnp.float32)
        return carry

    lax.fori_loop(0, BC, _fill_ones, 0)
    pltpu.sync_copy(vals_v.at[pl.ds(0, BC * 128)],
                    counts_sh.at[batch_v], add=True)
    plsc.subcore_barrier()

    pltpu.sync_copy(deg_sh.at[pl.ds(base, SL)], abuf)
    for i in range(SL // 16):
        d = abuf[pl.ds(i * 16, 16)]
        ii = plsc.bitcast(d, jnp.int32)
        ii = jnp.int32(MAGIC) - lax.shift_right_logical(ii, 1)
        r = plsc.bitcast(ii, jnp.float32)
        for _ in range(3):
            r = r * (1.5 - 0.5 * d * r * r)
        dinvbuf[pl.ds(i * 16, 16)] = r
        ubuf[pl.ds(i * 16, 16)] = r * ybuf[pl.ds(i * 16, 16)]
    pltpu.sync_copy(ubuf, u_sh.at[pl.ds(base, SL)])
    pltpu.sync_copy(ubuf, acc1_sh.at[pl.ds(base, SL)])
    plsc.subcore_barrier()

    _edge_round(u_sh, acc1_sh, src_v, dst_v, ulocal, accp, iota_v)
    plsc.subcore_barrier()

    pltpu.sync_copy(acc1_sh.at[pl.ds(base, SL)], abuf)
    for i in range(SL // 16):
        r = dinvbuf[pl.ds(i * 16, 16)]
        ubuf[pl.ds(i * 16, 16)] = r * r * abuf[pl.ds(i * 16, 16)]
    pltpu.sync_copy(ubuf, u_sh.at[pl.ds(base, SL)])
    pltpu.sync_copy(ubuf, acc2_sh.at[pl.ds(base, SL)])
    plsc.subcore_barrier()

    _edge_round(u_sh, acc2_sh, src_v, dst_v, ulocal, accp, iota_v)
    plsc.subcore_barrier()

    pltpu.sync_copy(acc2_sh.at[pl.ds(base, SL)], abuf)
    for i in range(SL // 16):
        pvals_v[pl.ds(i * 16, 16)] = (
            dinvbuf[pl.ds(i * 16, 16)] * abuf[pl.ds(i * 16, 16)])

    pltpu.sync_copy(pvals_v, sums_sh.at[batch_v], add=True)
    plsc.subcore_barrier()

    @pl.when((s == 0) & (c == 0))
    def _finalize():
        pltpu.sync_copy(sums_sh, sbuf)
        pltpu.sync_copy(counts_sh, cbuf)
        for i in range(NUM_GRAPHS // 16):
            sv = sbuf[pl.ds(i * 16, 16)]
            cv = cbuf[pl.ds(i * 16, 16)]
            obuf[pl.ds(i * 16, 16)] = sv / jnp.maximum(cv, 1.0)
            ocnt[pl.ds(i * 16, 16)] = cv
        pltpu.sync_copy(obuf, pooled_hbm)
        pltpu.sync_copy(ocnt, cnt_hbm)


_sc_call = functools.partial(
    pl.kernel,
    out_type=(jax.ShapeDtypeStruct((NUM_GRAPHS,), jnp.float32),
              jax.ShapeDtypeStruct((NUM_GRAPHS,), jnp.float32)),
    mesh=plsc.VectorSubcoreMesh(core_axis_name="c", subcore_axis_name="s",
                                num_cores=1),
    compiler_params=pltpu.CompilerParams(needs_layout_passes=False),
    scratch_types=[
        pltpu.VMEM((E_TILE,), jnp.int32),
        pltpu.VMEM((E_TILE,), jnp.int32),
        pltpu.VMEM((BC * 128,), jnp.float32),
        pltpu.VMEM((BC * 128,), jnp.int32),
        pltpu.VMEM((BC * 128,), jnp.float32),
        pltpu.VMEM((N_SH,), jnp.float32),
        pltpu.VMEM((N_SH,), jnp.float32),
        pltpu.VMEM((N_SH,), jnp.int32),
        pltpu.VMEM((SL,), jnp.float32),
        pltpu.VMEM((SL,), jnp.float32),
        pltpu.VMEM((SL,), jnp.float32),
        pltpu.VMEM((SL,), jnp.float32),
        pltpu.VMEM((128,), jnp.float32),
        pltpu.VMEM((G_SH,), jnp.float32),
        pltpu.VMEM((G_SH,), jnp.float32),
        pltpu.VMEM((NUM_GRAPHS,), jnp.float32),
        pltpu.VMEM((NUM_GRAPHS,), jnp.float32),
        pltpu.VMEM_SHARED((N_SH,), jnp.float32),
        pltpu.VMEM_SHARED((N_SH,), jnp.float32),
        pltpu.VMEM_SHARED((N_SH,), jnp.float32),
        pltpu.VMEM_SHARED((N_SH,), jnp.float32),
        pltpu.VMEM_SHARED((G_SH,), jnp.float32),
        pltpu.VMEM_SHARED((G_SH,), jnp.float32),
        pltpu.SemaphoreType.DMA,
    ],
)(_sc_body)


def kernel(x, edge_index, batch, W, b):
    y2d = pl.pallas_call(
        _matvec_body,
        out_shape=jax.ShapeDtypeStruct((N_NODES, 1), jnp.float32),
    )(x, W)
    y_pad = jnp.pad(y2d[:, 0], (0, N_SH - N_NODES))
    batch_p = jnp.pad(batch, (0, N_SH - N_NODES),
                      constant_values=DUMMY_G).reshape(NS, SL)
    batch_p = jnp.pad(batch_p, ((0, 0), (0, BC * 128 - SL)),
                      constant_values=DUMMY_G).reshape(NS, BC * 128)
    pooled, counts = _sc_call(edge_index[0], edge_index[1], batch_p, y_pad)
    out = pooled + b * (counts > 0.0)
    return out.reshape(NUM_GRAPHS, 1)
